# baseline (speedup 1.0000x reference)
import jax
import jax.numpy as jnp
from jax.experimental import pallas as pl


def _attn_pool_kernel(x_ref, w1q_ref, w1x_ref, b1_ref, w2_ref, o_ref):
    xb = x_ref[...]
    q = xb[0:1, :]
    qb = (jnp.dot(q, w1q_ref[...], preferred_element_type=jnp.float32)
          + b1_ref[...])
    pre = (jnp.dot(xb, w1x_ref[...], preferred_element_type=jnp.float32)
           + qb)
    h = pre * jax.nn.sigmoid(pre)
    e = jax.lax.dot_general(w2_ref[...], h, (((1,), (1,)), ((), ())),
                            preferred_element_type=jnp.float32)
    col = jax.lax.broadcasted_iota(jnp.int32, e.shape, 1)
    e = jnp.where(col == 0, -1e9, e)
    a = jnp.exp(e - jnp.max(e))
    ctx = jax.lax.dot_general(a, xb, (((1,), (0,)), ((), ())),
                              preferred_element_type=jnp.float32)
    o_ref[0] = ctx * (1.0 / jnp.sum(a))


def kernel(x, absorber_mask, batch, W1, b1, W2, b2):
    N, D = x.shape
    H = W1.shape[1]
    G = 100
    P = N // G
    W1q = W1[:D, :]
    W1x = W1[D:, :]
    b1r = b1.reshape(1, H)
    w2r = W2.reshape(1, H)
    return pl.pallas_call(
        _attn_pool_kernel,
        grid=(G,),
        in_specs=[
            pl.BlockSpec((P, D), lambda g: (g, 0)),
            pl.BlockSpec((D, H), lambda g: (0, 0)),
            pl.BlockSpec((D, H), lambda g: (0, 0)),
            pl.BlockSpec((1, H), lambda g: (0, 0)),
            pl.BlockSpec((1, H), lambda g: (0, 0)),
        ],
        out_specs=pl.BlockSpec((1, 1, D), lambda g: (g, 0, 0)),
        out_shape=jax.ShapeDtypeStruct((G, 1, D), jnp.float32),
    )(x, W1q, W1x, b1r, w2r).reshape(G, D)


# speedup vs baseline: 1.1427x; 1.1427x over previous
import jax
import jax.numpy as jnp
from jax.experimental import pallas as pl

_GPB = 4


def _attn_pool_kernel(x_ref, w1q_ref, w1x_ref, b1_ref, w2_ref, o_ref):
    P = x_ref.shape[0] // _GPB
    xb = x_ref[...]
    qs = jnp.concatenate([xb[j * P:j * P + 1] for j in range(_GPB)], axis=0)
    qb = (jnp.dot(qs, w1q_ref[...], preferred_element_type=jnp.float32)
          + b1_ref[...])
    pre = jnp.dot(xb, w1x_ref[...], preferred_element_type=jnp.float32)
    w2 = w2_ref[...]
    for j in range(_GPB):
        pj = pre[j * P:(j + 1) * P] + qb[j:j + 1]
        h = pj * jax.nn.sigmoid(pj)
        e = jax.lax.dot_general(w2, h, (((1,), (1,)), ((), ())),
                                preferred_element_type=jnp.float32)
        col = jax.lax.broadcasted_iota(jnp.int32, e.shape, 1)
        e = jnp.where(col == 0, -1e9, e)
        a = jnp.exp(e - jnp.max(e))
        ctx = jax.lax.dot_general(a, xb[j * P:(j + 1) * P],
                                  (((1,), (0,)), ((), ())),
                                  preferred_element_type=jnp.float32)
        o_ref[j] = ctx * (1.0 / jnp.sum(a))


def kernel(x, absorber_mask, batch, W1, b1, W2, b2):
    N, D = x.shape
    H = W1.shape[1]
    G = 100
    P = N // G
    W1q = W1[:D, :]
    W1x = W1[D:, :]
    b1r = b1.reshape(1, H)
    w2r = W2.reshape(1, H)
    return pl.pallas_call(
        _attn_pool_kernel,
        grid=(G // _GPB,),
        in_specs=[
            pl.BlockSpec((_GPB * P, D), lambda g: (g, 0)),
            pl.BlockSpec((D, H), lambda g: (0, 0)),
            pl.BlockSpec((D, H), lambda g: (0, 0)),
            pl.BlockSpec((1, H), lambda g: (0, 0)),
            pl.BlockSpec((1, H), lambda g: (0, 0)),
        ],
        out_specs=pl.BlockSpec((_GPB, 1, D), lambda g: (g, 0, 0)),
        out_shape=jax.ShapeDtypeStruct((G, 1, D), jnp.float32),
    )(x, W1q, W1x, b1r, w2r).reshape(G, D)
